# baseline (speedup 1.0000x reference)
import functools

import jax
import jax.numpy as jnp
from jax import lax
from jax.experimental import pallas as pl
from jax.experimental.pallas import tpu as pltpu
from jax.experimental.pallas import tpu_sc as plsc

_INPUT_DIMS = [185, 102, 108, 136, 51, 136, 1002]
_OUT_DIMS = [14, 11, 11, 12, 8, 12, 32]
_DEEP = 256
_B, _L = 1024, 50
_NF = 7
_S = _NF * _L
_ROWS = _B * _S

_VPAD = [(v + 7) // 8 * 8 for v in _INPUT_DIMS]
_OFFS = [0]
for _v in _VPAD[:-1]:
    _OFFS.append(_OFFS[-1] + _v)
_VTOT = _OFFS[-1] + _VPAD[-1]
_KPAD = 32

_NC, _NS = 2, 16
_NW = _NC * _NS
_NI = _ROWS // _NW
_CH = 112
_NREP = 32
_NCH = _NI // _CH
_NBUF = 4


def _tables_body(e_ref, w_ref, out_ref):
    for f in range(_NF):
        out_ref[_OFFS[f]:_OFFS[f] + _VPAD[f], :] = jnp.dot(
            e_ref[_OFFS[f]:_OFFS[f] + _VPAD[f], :],
            w_ref[f, :, :],
            preferred_element_type=jnp.float32,
        )


def _make_tables(Ep, Ws):
    return pl.pallas_call(
        _tables_body,
        grid=(_NREP,),
        in_specs=[
            pl.BlockSpec((_VTOT, _KPAD), lambda i: (0, 0)),
            pl.BlockSpec((_NF, _KPAD, _DEEP), lambda i: (0, 0, 0)),
        ],
        out_specs=pl.BlockSpec((_VTOT, _DEEP), lambda i: (i, 0)),
        out_shape=jax.ShapeDtypeStruct((_NREP * _VTOT, _DEEP), jnp.float32),
    )(Ep, Ws)


_sc_mesh = plsc.VectorSubcoreMesh(core_axis_name="c", subcore_axis_name="s")


@functools.partial(
    pl.kernel,
    mesh=_sc_mesh,
    out_type=jax.ShapeDtypeStruct((_ROWS, _DEEP), jnp.float32),
    scratch_types=(
        [pltpu.VMEM((_NI,), jnp.int32)]
        + [pltpu.VMEM((_CH, _DEEP), jnp.float32) for _ in range(_NBUF)]
        + [pltpu.SemaphoreType.DMA for _ in range(2 * _NBUF)]
    ),
)
def _sc_gather(table_hbm, idx_hbm, out_hbm, idx_v,
               buf0, buf1, buf2, buf3,
               gs0, gs1, gs2, gs3, ss0, ss1, ss2, ss3):
    bufs = [buf0, buf1, buf2, buf3]
    gsems = [gs0, gs1, gs2, gs3]
    ssems = [ss0, ss1, ss2, ss3]

    wid = lax.axis_index("s") * _NC + lax.axis_index("c")
    r_base = wid * _NI
    pltpu.sync_copy(idx_hbm.at[pl.ds(r_base, _NI)], idx_v)

    def _src(c):
        return table_hbm.at[idx_v.at[pl.ds(c * _CH, _CH)]]

    def _dst(c):
        return out_hbm.at[pl.ds(r_base + c * _CH, _CH)]

    def _fire(c, buf, sem):
        pltpu.async_copy(_src(c), buf, sem)

    def _gwait(c, buf, sem):
        pltpu.make_async_copy(_src(c), buf, sem).wait()

    def _store(c, buf, sem):
        pltpu.async_copy(buf, _dst(c), sem)

    def _swait(c, buf, sem):
        pltpu.make_async_copy(buf, _dst(c), sem).wait()

    for j in range(_NBUF - 1):
        _fire(j, bufs[j], gsems[j])

    @pl.loop(0, _NCH, step=_NBUF)
    def _(u):
        for j in range(_NBUF):
            jn = (j + _NBUF - 1) % _NBUF

            @pl.when(u + j + _NBUF - 1 < _NCH)
            def _():
                @pl.when(u + j - 1 >= 0)
                def _():
                    _swait(u + j - 1, bufs[jn], ssems[jn])

                _fire(u + j + _NBUF - 1, bufs[jn], gsems[jn])

            _gwait(u + j, bufs[j], gsems[j])
            _store(u + j, bufs[j], ssems[j])

    for j in range(_NBUF):
        _swait(_NCH - _NBUF + j, bufs[j], ssems[j])


def kernel(x1, x2, x3, x4, x5, x6, x7, E1, E2, E3, E4, E5, E6, E7,
           W1, W2, W3, W4, W5, W6, W7):
    xs = [x1, x2, x3, x4, x5, x6, x7]
    Es = [E1, E2, E3, E4, E5, E6, E7]
    Ws = [W1, W2, W3, W4, W5, W6, W7]

    Ep = jnp.concatenate(
        [jnp.pad(E, ((0, vp - v), (0, _KPAD - od)))
         for E, v, vp, od in zip(Es, _INPUT_DIMS, _VPAD, _OUT_DIMS)],
        axis=0,
    )
    Wstk = jnp.stack(
        [jnp.pad(W, ((0, _KPAD - od), (0, 0)))
         for W, od in zip(Ws, _OUT_DIMS)],
        axis=0,
    )
    table = _make_tables(Ep, Wstk)

    idx = jnp.concatenate(
        [x.astype(jnp.int32) + off for x, off in zip(xs, _OFFS)], axis=1
    ).reshape(-1)
    rep = (jnp.arange(_ROWS, dtype=jnp.int32) // _NI) % _NREP
    idx = idx + rep * _VTOT

    return _sc_gather(table, idx).reshape(_B, _S, _DEEP)


# speedup vs baseline: 1.0176x; 1.0176x over previous
import functools

import jax
import jax.numpy as jnp
from jax import lax
from jax.experimental import pallas as pl
from jax.experimental.pallas import tpu as pltpu
from jax.experimental.pallas import tpu_sc as plsc

_INPUT_DIMS = [185, 102, 108, 136, 51, 136, 1002]
_OUT_DIMS = [14, 11, 11, 12, 8, 12, 32]
_DEEP = 256
_B, _L = 1024, 50
_NF = 7
_S = _NF * _L
_ROWS = _B * _S

_VPAD = [(v + 7) // 8 * 8 for v in _INPUT_DIMS]
_OFFS = [0]
for _v in _VPAD[:-1]:
    _OFFS.append(_OFFS[-1] + _v)
_VTOT = _OFFS[-1] + _VPAD[-1]
_KPAD = 32

_NC, _NS = 2, 16
_NW = _NC * _NS
_NI = _ROWS // _NW
_CH = 224
_NREP = 8
_NCH = _NI // _CH
_NBUF = 2


def _tables_body(e_ref, w_ref, out_ref):
    for f in range(_NF):
        out_ref[_OFFS[f]:_OFFS[f] + _VPAD[f], :] = jnp.dot(
            e_ref[_OFFS[f]:_OFFS[f] + _VPAD[f], :],
            w_ref[f, :, :],
            preferred_element_type=jnp.float32,
        )


def _make_tables(Ep, Ws):
    return pl.pallas_call(
        _tables_body,
        grid=(_NREP,),
        in_specs=[
            pl.BlockSpec((_VTOT, _KPAD), lambda i: (0, 0)),
            pl.BlockSpec((_NF, _KPAD, _DEEP), lambda i: (0, 0, 0)),
        ],
        out_specs=pl.BlockSpec((_VTOT, _DEEP), lambda i: (i, 0)),
        out_shape=jax.ShapeDtypeStruct((_NREP * _VTOT, _DEEP), jnp.float32),
    )(Ep, Ws)


_sc_mesh = plsc.VectorSubcoreMesh(core_axis_name="c", subcore_axis_name="s")


@functools.partial(
    pl.kernel,
    mesh=_sc_mesh,
    out_type=jax.ShapeDtypeStruct((_ROWS, _DEEP), jnp.float32),
    scratch_types=(
        [pltpu.VMEM((_NI,), jnp.int32)]
        + [pltpu.VMEM((_CH, _DEEP), jnp.float32) for _ in range(_NBUF)]
        + [pltpu.SemaphoreType.DMA for _ in range(2 * _NBUF)]
    ),
)
def _sc_gather(table_hbm, idx_hbm, out_hbm, *scr):
    idx_v = scr[0]
    bufs = scr[1:1 + _NBUF]
    gsems = scr[1 + _NBUF:1 + 2 * _NBUF]
    ssems = scr[1 + 2 * _NBUF:1 + 3 * _NBUF]

    wid = lax.axis_index("s") * _NC + lax.axis_index("c")
    r_base = wid * _NI
    pltpu.sync_copy(idx_hbm.at[pl.ds(r_base, _NI)], idx_v)

    def _src(c):
        return table_hbm.at[idx_v.at[pl.ds(c * _CH, _CH)]]

    def _dst(c):
        return out_hbm.at[pl.ds(r_base + c * _CH, _CH)]

    def _fire(c, buf, sem):
        pltpu.async_copy(_src(c), buf, sem)

    def _gwait(c, buf, sem):
        pltpu.make_async_copy(_src(c), buf, sem).wait()

    def _store(c, buf, sem):
        pltpu.async_copy(buf, _dst(c), sem)

    def _swait(c, buf, sem):
        pltpu.make_async_copy(buf, _dst(c), sem).wait()

    for j in range(_NBUF - 1):
        _fire(j, bufs[j], gsems[j])

    @pl.loop(0, _NCH, step=_NBUF)
    def _(u):
        for j in range(_NBUF):
            jn = (j + _NBUF - 1) % _NBUF

            @pl.when(u + j + _NBUF - 1 < _NCH)
            def _():
                @pl.when(u + j - 1 >= 0)
                def _():
                    _swait(u + j - 1, bufs[jn], ssems[jn])

                _fire(u + j + _NBUF - 1, bufs[jn], gsems[jn])

            _gwait(u + j, bufs[j], gsems[j])
            _store(u + j, bufs[j], ssems[j])

    for j in range(_NBUF):
        _swait(_NCH - _NBUF + j, bufs[j], ssems[j])


def kernel(x1, x2, x3, x4, x5, x6, x7, E1, E2, E3, E4, E5, E6, E7,
           W1, W2, W3, W4, W5, W6, W7):
    xs = [x1, x2, x3, x4, x5, x6, x7]
    Es = [E1, E2, E3, E4, E5, E6, E7]
    Ws = [W1, W2, W3, W4, W5, W6, W7]

    Ep = jnp.concatenate(
        [jnp.pad(E, ((0, vp - v), (0, _KPAD - od)))
         for E, v, vp, od in zip(Es, _INPUT_DIMS, _VPAD, _OUT_DIMS)],
        axis=0,
    )
    Wstk = jnp.stack(
        [jnp.pad(W, ((0, _KPAD - od), (0, 0)))
         for W, od in zip(Ws, _OUT_DIMS)],
        axis=0,
    )
    table = _make_tables(Ep, Wstk)

    idx = jnp.concatenate(
        [x.astype(jnp.int32) + off for x, off in zip(xs, _OFFS)], axis=1
    ).reshape(-1)
    rep = (jnp.arange(_ROWS, dtype=jnp.int32) // _NI) % _NREP
    idx = idx + rep * _VTOT

    return _sc_gather(table, idx).reshape(_B, _S, _DEEP)
